# baseline (speedup 1.0000x reference)
import functools

import jax
import jax.numpy as jnp
from jax import lax
from jax.experimental import pallas as pl
from jax.experimental.pallas import tpu as pltpu
from jax.experimental.pallas import tpu_sc as plsc

_NC = 2
_NS = 16
_NW = _NC * _NS
_L = 16

_B = 1024
_SUB = 128
_NSUB = _B // _SUB

_R2 = float(1.0 / (2.0 ** 0.5))
_R3 = float(1.0 / (3.0 ** 0.5))
_R6 = float(1.0 / (6.0 ** 0.5))
_PRE = float(2.0 ** -0.25)
_S0 = float((2.0 / 3.0) ** 0.5)
_S6 = float(1.0 / (3.0 ** 0.5))


def _tp_accum(x, y):
    s = c1 = c2 = c3 = m2 = m1 = q0 = p1 = p2 = None

    def add(acc, v):
        return v if acc is None else acc + v

    for g in range(3):
        a, b, c = x[3 * g], x[3 * g + 1], x[3 * g + 2]
        d, e, f = y[3 * g], y[3 * g + 1], y[3 * g + 2]
        ad = a * d
        ae = a * e
        af = a * f
        bd = b * d
        be = b * e
        bf = b * f
        cd = c * d
        ce = c * e
        cf = c * f
        s = add(s, ad + be + cf)
        c1 = add(c1, bf - ce)
        c2 = add(c2, cd - af)
        c3 = add(c3, ae - bd)
        m2 = add(m2, ae + bd)
        m1 = add(m1, bf + ce)
        q0 = add(q0, cf + cf - ad - be)
        p1 = add(p1, af + cd)
        p2 = add(p2, ad - be)
    return s, c1, c2, c3, m2, m1, q0, p1, p2


def _edge_body(tab2d, rowi, coli, out,
               idxr0, idxc0, x0, y0, o0,
               idxr1, idxc1, x1, y1, o1, sem0, sem1, osem0, osem1):
    wid = lax.axis_index("s") * _NC + lax.axis_index("c")
    n_chunks = rowi.shape[0] // _NSUB
    my_count = (n_chunks - wid + _NW - 1) // _NW
    tab = tab2d

    lane = lax.iota(jnp.int32, _L)
    kfull = [jnp.full((_L,), k, jnp.int32) for k in range(9)]
    bufs = ((idxr0, idxc0, x0, y0, o0, sem0, osem0),
            (idxr1, idxc1, x1, y1, o1, sem1, osem1))

    def out_slab(c, o_v, osem):
        return pltpu.make_async_copy(
            o_v, out.at[:, pl.ds(c * (_B // _SUB), _B // _SUB), :], osem)

    def fire(c, buf):
        idxr, idxc, x_v, y_v, _, sem, _osem = buf
        pltpu.sync_copy(rowi.at[pl.ds(c * _NSUB, _NSUB)], idxr)
        pltpu.sync_copy(coli.at[pl.ds(c * _NSUB, _NSUB)], idxc)
        for j in range(_NSUB):
            pltpu.async_copy(
                tab.at[idxr.at[j]], x_v.at[pl.ds(j * _SUB, _SUB)], sem)
            pltpu.async_copy(
                tab.at[idxc.at[j]], y_v.at[pl.ds(j * _SUB, _SUB)], sem)

    def drain_compute(c, k, buf):
        idxr, idxc, x_v, y_v, o_v, sem, osem = buf
        for j in range(_NSUB):
            pltpu.make_async_copy(
                tab.at[idxr.at[j]], x_v.at[pl.ds(j * _SUB, _SUB)], sem).wait()
            pltpu.make_async_copy(
                tab.at[idxc.at[j]], y_v.at[pl.ds(j * _SUB, _SUB)], sem).wait()

        @pl.when(k >= 2)
        def _():
            out_slab(c, o_v, osem).wait()

        for srow in range(_B // _SUB):
            @plsc.parallel_loop(0, _SUB // _L, unroll=2)
            def grp(g):
                r = lane + (srow * _SUB) + g * _L
                x = [plsc.load_gather(x_v, [r, kfull[k]]) for k in range(9)]
                y = [plsc.load_gather(y_v, [r, kfull[k]]) for k in range(9)]
                s, c1, c2, c3, m2, m1, q0, p1, p2 = _tp_accum(x, y)
                outs = (s * _R3, c1 * _R2, c2 * _R2, c3 * _R2,
                        m2 * _R2, m1 * _R2, q0 * _R6, p1 * _R2, p2 * _R2)
                for j, v in enumerate(outs):
                    o_v[j, srow, pl.ds(g * _L, _L)] = v

        pltpu.async_copy(
            o_v, out.at[:, pl.ds(c * (_B // _SUB), _B // _SUB), :], osem)

    n_pairs = (my_count + 1) // 2

    @pl.when(my_count > 0)
    def _():
        fire(wid, bufs[0])

    def pair_body(i, _):
        k0 = 2 * i
        k1 = k0 + 1
        k2 = k0 + 2

        @pl.when(k1 < my_count)
        def _():
            fire(wid + k1 * _NW, bufs[1])

        drain_compute(wid + k0 * _NW, k0, bufs[0])

        @pl.when(k2 < my_count)
        def _():
            fire(wid + k2 * _NW, bufs[0])

        @pl.when(k1 < my_count)
        def _():
            drain_compute(wid + k1 * _NW, k1, bufs[1])

        return 0

    lax.fori_loop(0, n_pairs, pair_body, 0)

    @pl.when(my_count >= 1)
    def _():
        out_slab(wid, bufs[0][4], bufs[0][6]).wait()

    @pl.when(my_count >= 2)
    def _():
        out_slab(wid, bufs[1][4], bufs[1][6]).wait()


def _edge_tp(tab2d, rows2d, cols2d, n_edges):
    mesh = plsc.VectorSubcoreMesh(
        core_axis_name="c", subcore_axis_name="s",
        num_cores=_NC, num_subcores=_NS)
    buf_types = [
        pltpu.VMEM((_NSUB, _SUB), jnp.int32),
        pltpu.VMEM((_NSUB, _SUB), jnp.int32),
        pltpu.VMEM((_B, _L), jnp.float32),
        pltpu.VMEM((_B, _L), jnp.float32),
        pltpu.VMEM((9, _B // _SUB, _SUB), jnp.float32),
    ]
    f = pl.kernel(
        _edge_body,
        out_type=jax.ShapeDtypeStruct((9, n_edges // _SUB, _SUB),
                                      jnp.float32),
        mesh=mesh,
        scratch_types=buf_types + buf_types + [
            pltpu.SemaphoreType.DMA,
            pltpu.SemaphoreType.DMA,
            pltpu.SemaphoreType.DMA,
            pltpu.SemaphoreType.DMA,
        ],
        compiler_params=pltpu.CompilerParams(
            needs_layout_passes=False, use_tc_tiling_on_sc=False),
    )
    return f(tab2d, rows2d, cols2d)


def _ii_body(nf_ref, out_ref):
    out_ref[...] = jnp.concatenate(
        [nf_ref[:, 0:1], nf_ref[:, 10:15]], axis=-1)


def _ii_extract(node_features):
    n = node_features.shape[0]
    blk = 10000
    return pl.pallas_call(
        _ii_body,
        out_shape=jax.ShapeDtypeStruct((n, 6), jnp.float32),
        grid=(n // blk,),
        in_specs=[pl.BlockSpec((blk, 15), lambda i: (i, 0))],
        out_specs=pl.BlockSpec((blk, 6), lambda i: (i, 0)),
    )(node_features)


@jax.jit
def kernel(node_features, hessian_off_diag_layout):
    n_edges = hessian_off_diag_layout.shape[0]
    n_nodes = node_features.shape[0]
    tab2d = jnp.pad(node_features[:, 1:10], ((0, 0), (0, 7)))
    rows2d = hessian_off_diag_layout[:, 0].reshape(n_edges // _SUB, _SUB)
    cols2d = hessian_off_diag_layout[:, 1].reshape(n_edges // _SUB, _SUB)

    out_ii = _ii_extract(node_features)
    planes = _edge_tp(tab2d, rows2d, cols2d, n_edges)
    out_ij = planes.reshape(9, n_edges).T
    return (out_ii, out_ij)


# speedup vs baseline: 1.0914x; 1.0914x over previous
import jax
import jax.numpy as jnp
from jax import lax
from jax.experimental import pallas as pl
from jax.experimental.pallas import tpu as pltpu
from jax.experimental.pallas import tpu_sc as plsc

_NC = 2
_NS = 16
_NW = _NC * _NS
_L = 16

_B = 1024
_SUB = 128
_NSUB = _B // _SUB

_R2 = float(1.0 / (2.0 ** 0.5))
_R3 = float(1.0 / (3.0 ** 0.5))
_R6 = float(1.0 / (6.0 ** 0.5))


def _tp_accum(x, y):
    s = c1 = c2 = c3 = m2 = m1 = q0 = p1 = p2 = None

    def add(acc, v):
        return v if acc is None else acc + v

    for g in range(3):
        a, b, c = x[3 * g], x[3 * g + 1], x[3 * g + 2]
        d, e, f = y[3 * g], y[3 * g + 1], y[3 * g + 2]
        ad = a * d
        ae = a * e
        af = a * f
        bd = b * d
        be = b * e
        bf = b * f
        cd = c * d
        ce = c * e
        cf = c * f
        s = add(s, ad + be + cf)
        c1 = add(c1, bf - ce)
        c2 = add(c2, cd - af)
        c3 = add(c3, ae - bd)
        m2 = add(m2, ae + bd)
        m1 = add(m1, bf + ce)
        q0 = add(q0, cf + cf - ad - be)
        p1 = add(p1, af + cd)
        p2 = add(p2, ad - be)
    return s, c1, c2, c3, m2, m1, q0, p1, p2


def _edge_body(tab2d, rowi, coli, out,
               idxr0, idxc0, x0, y0, o0,
               idxr1, idxc1, x1, y1, o1, sem0, sem1, osem0, osem1):
    wid = lax.axis_index("s") * _NC + lax.axis_index("c")
    n_chunks = rowi.shape[0] // _NSUB
    my_count = (n_chunks - wid + _NW - 1) // _NW
    tab = tab2d

    lane = lax.iota(jnp.int32, _L)
    kfull = [jnp.full((_L,), k, jnp.int32) for k in range(9)]
    bufs = ((idxr0, idxc0, x0, y0, o0, sem0, osem0),
            (idxr1, idxc1, x1, y1, o1, sem1, osem1))

    def out_slab(c, o_v, osem):
        return pltpu.make_async_copy(
            o_v, out.at[:, pl.ds(c * (_B // _SUB), _B // _SUB), :], osem)

    def fire(c, buf):
        idxr, idxc, x_v, y_v, _, sem, _osem = buf
        pltpu.sync_copy(rowi.at[pl.ds(c * _NSUB, _NSUB)], idxr)
        pltpu.sync_copy(coli.at[pl.ds(c * _NSUB, _NSUB)], idxc)
        for j in range(_NSUB):
            pltpu.async_copy(
                tab.at[idxr.at[j]], x_v.at[pl.ds(j * _SUB, _SUB)], sem)
            pltpu.async_copy(
                tab.at[idxc.at[j]], y_v.at[pl.ds(j * _SUB, _SUB)], sem)

    def drain_compute(c, k, buf):
        idxr, idxc, x_v, y_v, o_v, sem, osem = buf
        for j in range(_NSUB):
            pltpu.make_async_copy(
                tab.at[idxr.at[j]], x_v.at[pl.ds(j * _SUB, _SUB)], sem).wait()
            pltpu.make_async_copy(
                tab.at[idxc.at[j]], y_v.at[pl.ds(j * _SUB, _SUB)], sem).wait()

        @pl.when(k >= 2)
        def _():
            out_slab(c, o_v, osem).wait()

        for srow in range(_B // _SUB):
            @plsc.parallel_loop(0, _SUB // _L, unroll=4)
            def grp(g):
                r = lane + (srow * _SUB) + g * _L
                x = [plsc.load_gather(x_v, [r, kfull[k]]) for k in range(9)]
                y = [plsc.load_gather(y_v, [r, kfull[k]]) for k in range(9)]
                s, c1, c2, c3, m2, m1, q0, p1, p2 = _tp_accum(x, y)
                outs = (s * _R3, c1 * _R2, c2 * _R2, c3 * _R2,
                        m2 * _R2, m1 * _R2, q0 * _R6, p1 * _R2, p2 * _R2)
                for j, v in enumerate(outs):
                    o_v[j, srow, pl.ds(g * _L, _L)] = v

        pltpu.async_copy(
            o_v, out.at[:, pl.ds(c * (_B // _SUB), _B // _SUB), :], osem)

    n_pairs = (my_count + 1) // 2

    @pl.when(my_count > 0)
    def _():
        fire(wid, bufs[0])

    def pair_body(i, _):
        k0 = 2 * i
        k1 = k0 + 1
        k2 = k0 + 2

        @pl.when(k1 < my_count)
        def _():
            fire(wid + k1 * _NW, bufs[1])

        drain_compute(wid + k0 * _NW, k0, bufs[0])

        @pl.when(k2 < my_count)
        def _():
            fire(wid + k2 * _NW, bufs[0])

        @pl.when(k1 < my_count)
        def _():
            drain_compute(wid + k1 * _NW, k1, bufs[1])

        return 0

    lax.fori_loop(0, n_pairs, pair_body, 0)

    @pl.when(my_count >= 1)
    def _():
        out_slab(wid, bufs[0][4], bufs[0][6]).wait()

    @pl.when(my_count >= 2)
    def _():
        out_slab(wid, bufs[1][4], bufs[1][6]).wait()


def _edge_tp(tab2d, rows2d, cols2d, n_edges):
    mesh = plsc.VectorSubcoreMesh(
        core_axis_name="c", subcore_axis_name="s",
        num_cores=_NC, num_subcores=_NS)
    buf_types = [
        pltpu.VMEM((_NSUB, _SUB), jnp.int32),
        pltpu.VMEM((_NSUB, _SUB), jnp.int32),
        pltpu.VMEM((_B, _L), jnp.float32),
        pltpu.VMEM((_B, _L), jnp.float32),
        pltpu.VMEM((9, _B // _SUB, _SUB), jnp.float32),
    ]
    f = pl.kernel(
        _edge_body,
        out_type=jax.ShapeDtypeStruct((9, n_edges // _SUB, _SUB),
                                      jnp.float32),
        mesh=mesh,
        scratch_types=buf_types + buf_types + [
            pltpu.SemaphoreType.DMA,
            pltpu.SemaphoreType.DMA,
            pltpu.SemaphoreType.DMA,
            pltpu.SemaphoreType.DMA,
        ],
        compiler_params=pltpu.CompilerParams(
            needs_layout_passes=False, use_tc_tiling_on_sc=False),
    )
    return f(tab2d, rows2d, cols2d)


def _ii_body(nf_ref, out_ref):
    out_ref[...] = jnp.concatenate(
        [nf_ref[:, 0:1], nf_ref[:, 10:15]], axis=-1)


def _ii_extract(node_features):
    n = node_features.shape[0]
    blk = 10000
    return pl.pallas_call(
        _ii_body,
        out_shape=jax.ShapeDtypeStruct((n, 6), jnp.float32),
        grid=(n // blk,),
        in_specs=[pl.BlockSpec((blk, 15), lambda i: (i, 0))],
        out_specs=pl.BlockSpec((blk, 6), lambda i: (i, 0)),
    )(node_features)


@jax.jit
def kernel(node_features, hessian_off_diag_layout):
    n_edges = hessian_off_diag_layout.shape[0]
    tab2d = jnp.pad(node_features[:, 1:10], ((0, 0), (0, 7)))
    rows2d = hessian_off_diag_layout[:, 0].reshape(n_edges // _SUB, _SUB)
    cols2d = hessian_off_diag_layout[:, 1].reshape(n_edges // _SUB, _SUB)

    out_ii = _ii_extract(node_features)
    planes = _edge_tp(tab2d, rows2d, cols2d, n_edges)
    out_ij = planes.reshape(9, n_edges).T
    return (out_ii, out_ij)
